# Initial kernel scaffold; baseline (speedup 1.0000x reference)
import jax
import jax.numpy as jnp
from jax.experimental import pallas as pl


def kernel(x, Wg, bg, W1, b1, W2, b2):
    raise NotImplementedError("write your pallas kernel here")


# speedup vs baseline: 7.3316x; 7.3316x over previous
import functools

import jax
import jax.numpy as jnp
from jax import lax
from jax.experimental import pallas as pl
from jax.experimental.pallas import tpu as pltpu
from jax.experimental.pallas import tpu_sc as plsc

N = 2048
D = 768
F = 768
E = 64
TILE = 256
NUM_TILES = N // TILE
NUM_PAIRS = NUM_TILES + E - 1

_NC = 2
_NS = 16
_NW = _NC * _NS
_ROWS_PER_WORKER = N // _NW


def _router_body(x_ref, wg_ref, bg_ref, idx_ref):
    logits = jnp.dot(x_ref[:], wg_ref[:], preferred_element_type=jnp.float32)
    logits = logits + bg_ref[:]
    m = jnp.max(logits, axis=1, keepdims=True)
    col = lax.broadcasted_iota(jnp.int32, logits.shape, 1)
    cand = jnp.where(logits == m, col, jnp.int32(E))
    idx_ref[:] = jnp.min(cand, axis=1, keepdims=True)


def _route(x_flat, Wg, bg):
    return pl.pallas_call(
        _router_body,
        out_shape=jax.ShapeDtypeStruct((N, 1), jnp.int32),
    )(x_flat, Wg, bg.reshape(1, E))[:, 0]


def _gather_rows(table, indices):
    mesh = plsc.VectorSubcoreMesh(core_axis_name="c", subcore_axis_name="s")

    @functools.partial(
        pl.kernel,
        out_type=jax.ShapeDtypeStruct((N, D), jnp.float32),
        mesh=mesh,
        scratch_types=[
            pltpu.VMEM((_ROWS_PER_WORKER,), jnp.int32),
            pltpu.VMEM((_ROWS_PER_WORKER, D), jnp.float32),
            pltpu.SemaphoreType.DMA,
        ],
    )
    def k(table_hbm, idx_hbm, out_hbm, idx_v, rows_v, sem):
        wid = lax.axis_index("s") * _NC + lax.axis_index("c")
        base = wid * _ROWS_PER_WORKER
        pltpu.sync_copy(idx_hbm.at[pl.ds(base, _ROWS_PER_WORKER)], idx_v)
        pltpu.async_copy(table_hbm.at[idx_v], rows_v, sem).wait()
        pltpu.sync_copy(rows_v, out_hbm.at[pl.ds(base, _ROWS_PER_WORKER)])

    return k(table, indices)


def _pair_schedule(idx):
    counts = jnp.bincount(idx, length=E)
    ends = jnp.cumsum(counts)
    starts = ends - counts
    t0 = starts // TILE
    t1 = (ends - 1) // TILE
    tcol = jnp.arange(NUM_TILES)[None, :]
    valid = (counts[:, None] > 0) & (tcol >= t0[:, None]) & (tcol <= t1[:, None])
    big = jnp.int32(NUM_TILES * E)
    key = jnp.where(valid, tcol * E + jnp.arange(E)[:, None], big)
    k = jnp.sort(key.ravel())[:NUM_PAIRS]
    isvalid = k < big
    tid = jnp.where(isvalid, k // E, NUM_TILES - 1).astype(jnp.int32)
    eid = jnp.where(isvalid, k % E, 0).astype(jnp.int32)
    g_start = jnp.maximum(starts[eid], tid * TILE)
    g_end = jnp.minimum(ends[eid], (tid + 1) * TILE)
    g_start = jnp.where(isvalid, g_start, 0).astype(jnp.int32)
    g_end = jnp.where(isvalid, g_end, 0).astype(jnp.int32)
    first = jnp.concatenate(
        [jnp.ones((1,), jnp.int32), (tid[1:] != tid[:-1]).astype(jnp.int32)]
    )
    return tid, eid, g_start, g_end, first


def _ffn_body(tid_ref, eid_ref, s_ref, e_ref, f_ref,
              x_ref, w1_ref, b1_ref, w2_ref, b2_ref, o_ref):
    i = pl.program_id(0)

    @pl.when(f_ref[i] == 1)
    def _():
        o_ref[:] = jnp.zeros_like(o_ref)

    h = jnp.dot(x_ref[:], w1_ref[0], preferred_element_type=jnp.float32)
    h = h + b1_ref[0]
    h = 0.5 * h * (1.0 + lax.erf(h * 0.7071067811865476))
    y = jnp.dot(h, w2_ref[0], preferred_element_type=jnp.float32)
    y = y + b2_ref[0]
    row = tid_ref[i] * TILE + lax.broadcasted_iota(jnp.int32, (TILE, 1), 0)
    mask = (row >= s_ref[i]) & (row < e_ref[i])
    o_ref[:] = jnp.where(mask, y, o_ref[:])


def _grouped_ffn(x_sorted, W1, b1, W2, b2, tid, eid, g_start, g_end, first):
    grid_spec = pltpu.PrefetchScalarGridSpec(
        num_scalar_prefetch=5,
        grid=(NUM_PAIRS,),
        in_specs=[
            pl.BlockSpec((TILE, D), lambda i, t, e, s, g, f: (t[i], 0)),
            pl.BlockSpec((1, D, F), lambda i, t, e, s, g, f: (e[i], 0, 0)),
            pl.BlockSpec((1, 1, F), lambda i, t, e, s, g, f: (e[i], 0, 0)),
            pl.BlockSpec((1, F, D), lambda i, t, e, s, g, f: (e[i], 0, 0)),
            pl.BlockSpec((1, 1, D), lambda i, t, e, s, g, f: (e[i], 0, 0)),
        ],
        out_specs=pl.BlockSpec((TILE, D), lambda i, t, e, s, g, f: (t[i], 0)),
    )
    return pl.pallas_call(
        _ffn_body,
        grid_spec=grid_spec,
        out_shape=jax.ShapeDtypeStruct((N, D), jnp.float32),
        compiler_params=pltpu.CompilerParams(
            dimension_semantics=("arbitrary",),
        ),
    )(tid, eid, g_start, g_end, first, x_sorted,
      W1, b1.reshape(E, 1, F), W2, b2.reshape(E, 1, D))


def kernel(x, Wg, bg, W1, b1, W2, b2):
    B, T, _ = x.shape
    x_flat = x.reshape(N, D)
    idx = _route(x_flat, Wg, bg)
    perm = jnp.argsort(idx)
    inv_perm = jnp.argsort(perm).astype(jnp.int32)
    tid, eid, g_start, g_end, first = _pair_schedule(idx)
    x_sorted = _gather_rows(x_flat, perm.astype(jnp.int32))
    out_sorted = _grouped_ffn(x_sorted, W1, b1, W2, b2,
                              tid, eid, g_start, g_end, first)
    out = _gather_rows(out_sorted, inv_perm)
    return out.reshape(B, T, D)
